# baseline (speedup 1.0000x reference)
import functools

import jax
import jax.numpy as jnp
from jax.experimental import pallas as pl

_NEG_INF = float("-inf")
_BIG = 1 << 30
_LANES = 128


def _dlr_body(x_ref, y_ref, o_ref):
    x = x_ref[...]
    yv = y_ref[...]
    R, C = x.shape
    n_full = C // _LANES
    tail = C - n_full * _LANES

    col = jax.lax.broadcasted_iota(jnp.int32, (R, _LANES), 1)
    neg = jnp.full((R, _LANES), _NEG_INF, dtype=jnp.float32)

    ch = x[:, :_LANES]
    m0, m1, m2 = ch, neg, neg
    cacc = jnp.where(col == yv, ch, _NEG_INF)

    def insert(ch, k, m0, m1, m2, cacc):
        cacc = jnp.maximum(cacc, jnp.where(col == yv - (k * _LANES), ch, _NEG_INF))
        t1 = jnp.minimum(m0, ch)
        m0 = jnp.maximum(m0, ch)
        t2 = jnp.minimum(m1, t1)
        m1 = jnp.maximum(m1, t1)
        m2 = jnp.maximum(m2, t2)
        return m0, m1, m2, cacc

    for k in range(1, n_full):
        m0, m1, m2, cacc = insert(x[:, k * _LANES:(k + 1) * _LANES], k, m0, m1, m2, cacc)

    cht = jnp.concatenate([x[:, n_full * _LANES:], neg[:, : _LANES - tail]], axis=1)
    m0, m1, m2, cacc = insert(cht, n_full, m0, m1, m2, cacc)

    c = jnp.max(cacc, axis=1, keepdims=True)

    u = jnp.concatenate([m0, m1, m2], axis=1)
    ucol = jax.lax.broadcasted_iota(jnp.int32, u.shape, 1)
    p0 = jnp.max(u, axis=1, keepdims=True)
    a0 = jnp.min(jnp.where(u == p0, ucol, _BIG), axis=1, keepdims=True)
    u1 = jnp.where(ucol == a0, _NEG_INF, u)
    p1 = jnp.max(u1, axis=1, keepdims=True)
    a1 = jnp.min(jnp.where(u1 == p1, ucol, _BIG), axis=1, keepdims=True)
    u2 = jnp.where(ucol == a1, _NEG_INF, u1)
    p2 = jnp.max(u2, axis=1, keepdims=True)

    target = jnp.where(c == p0, p1, p0)
    o_ref[...] = (target - c) / (p0 - p2)


@functools.partial(jax.jit, static_argnames=("block_rows",))
def _dlr_tc(prediction, y, block_rows=256):
    n, c = prediction.shape
    y2 = y.reshape(n, 1)
    out = pl.pallas_call(
        _dlr_body,
        grid=(n // block_rows,),
        in_specs=[
            pl.BlockSpec((block_rows, c), lambda i: (i, 0)),
            pl.BlockSpec((block_rows, 1), lambda i: (i, 0)),
        ],
        out_specs=pl.BlockSpec((block_rows, 1), lambda i: (i, 0)),
        out_shape=jax.ShapeDtypeStruct((n, 1), jnp.float32),
    )(prediction, y2)
    return out.reshape(n)


def kernel(prediction, y):
    return _dlr_tc(prediction, y)


# speedup vs baseline: 1.1529x; 1.1529x over previous
import functools

import jax
import jax.numpy as jnp
from jax.experimental import pallas as pl

_NEG_INF = float("-inf")
_BIG = 1 << 30
_LANES = 128


def _dlr_body(x_ref, y_ref, o_ref):
    x = x_ref[...]
    yv = y_ref[...].reshape(-1, 1)
    R, C = x.shape
    n_full = C // _LANES
    tail = C - n_full * _LANES

    col = jax.lax.broadcasted_iota(jnp.int32, (R, _LANES), 1)
    neg = jnp.full((R, _LANES), _NEG_INF, dtype=jnp.float32)

    ch = x[:, :_LANES]
    m0, m1, m2 = ch, neg, neg
    cacc = jnp.where(col == yv, ch, _NEG_INF)

    def insert(ch, k, m0, m1, m2, cacc):
        cacc = jnp.maximum(cacc, jnp.where(col == yv - (k * _LANES), ch, _NEG_INF))
        t1 = jnp.minimum(m0, ch)
        m0 = jnp.maximum(m0, ch)
        t2 = jnp.minimum(m1, t1)
        m1 = jnp.maximum(m1, t1)
        m2 = jnp.maximum(m2, t2)
        return m0, m1, m2, cacc

    for k in range(1, n_full):
        m0, m1, m2, cacc = insert(x[:, k * _LANES:(k + 1) * _LANES], k, m0, m1, m2, cacc)

    cht = jnp.concatenate([x[:, n_full * _LANES:], neg[:, : _LANES - tail]], axis=1)
    m0, m1, m2, cacc = insert(cht, n_full, m0, m1, m2, cacc)

    c = jnp.max(cacc, axis=1, keepdims=True)

    u = jnp.concatenate([m0, m1, m2], axis=1)
    ucol = jax.lax.broadcasted_iota(jnp.int32, u.shape, 1)
    p0 = jnp.max(u, axis=1, keepdims=True)
    a0 = jnp.min(jnp.where(u == p0, ucol, _BIG), axis=1, keepdims=True)
    u1 = jnp.where(ucol == a0, _NEG_INF, u)
    p1 = jnp.max(u1, axis=1, keepdims=True)
    a1 = jnp.min(jnp.where(u1 == p1, ucol, _BIG), axis=1, keepdims=True)
    u2 = jnp.where(ucol == a1, _NEG_INF, u1)
    p2 = jnp.max(u2, axis=1, keepdims=True)

    target = jnp.where(c == p0, p1, p0)
    o_ref[...] = ((target - c) / (p0 - p2)).reshape(-1)


@functools.partial(jax.jit, static_argnames=("block_rows",))
def _dlr_tc(prediction, y, block_rows=256):
    n, c = prediction.shape
    return pl.pallas_call(
        _dlr_body,
        grid=(n // block_rows,),
        in_specs=[
            pl.BlockSpec((block_rows, c), lambda i: (i, 0)),
            pl.BlockSpec((block_rows,), lambda i: (i,)),
        ],
        out_specs=pl.BlockSpec((block_rows,), lambda i: (i,)),
        out_shape=jax.ShapeDtypeStruct((n,), jnp.float32),
    )(prediction, y)


def kernel(prediction, y):
    return _dlr_tc(prediction, y)


# speedup vs baseline: 1.3467x; 1.1681x over previous
import functools

import jax
import jax.numpy as jnp
from jax.experimental import pallas as pl

_NEG_INF = float("-inf")
_BIG = 1 << 30
_LANES = 128


def _dlr_body(x_ref, y_ref, o_ref):
    x = x_ref[...]
    yv = y_ref[...].reshape(-1, 1)
    R, C = x.shape
    n_full = C // _LANES
    tail = C - n_full * _LANES

    col = jax.lax.broadcasted_iota(jnp.int32, (R, _LANES), 1)
    neg = jnp.full((R, _LANES), _NEG_INF, dtype=jnp.float32)

    ch = x[:, :_LANES]
    m0, m1, m2 = ch, neg, neg
    cacc = jnp.where(col == yv, ch, _NEG_INF)

    def insert(ch, k, m0, m1, m2, cacc):
        cacc = jnp.maximum(cacc, jnp.where(col == yv - (k * _LANES), ch, _NEG_INF))
        t1 = jnp.minimum(m0, ch)
        m0 = jnp.maximum(m0, ch)
        t2 = jnp.minimum(m1, t1)
        m1 = jnp.maximum(m1, t1)
        m2 = jnp.maximum(m2, t2)
        return m0, m1, m2, cacc

    for k in range(1, n_full):
        m0, m1, m2, cacc = insert(x[:, k * _LANES:(k + 1) * _LANES], k, m0, m1, m2, cacc)

    cht = jnp.concatenate([x[:, n_full * _LANES:], neg[:, : _LANES - tail]], axis=1)
    m0, m1, m2, cacc = insert(cht, n_full, m0, m1, m2, cacc)

    c = jnp.max(cacc, axis=1, keepdims=True)

    u = jnp.concatenate([m0, m1, m2], axis=1)
    ucol = jax.lax.broadcasted_iota(jnp.int32, u.shape, 1)
    p0 = jnp.max(u, axis=1, keepdims=True)
    a0 = jnp.min(jnp.where(u == p0, ucol, _BIG), axis=1, keepdims=True)
    u1 = jnp.where(ucol == a0, _NEG_INF, u)
    p1 = jnp.max(u1, axis=1, keepdims=True)
    a1 = jnp.min(jnp.where(u1 == p1, ucol, _BIG), axis=1, keepdims=True)
    u2 = jnp.where(ucol == a1, _NEG_INF, u1)
    p2 = jnp.max(u2, axis=1, keepdims=True)

    target = jnp.where(c == p0, p1, p0)
    o_ref[...] = ((target - c) / (p0 - p2)).reshape(-1)


@functools.partial(jax.jit, static_argnames=("block_rows",))
def _dlr_tc(prediction, y, block_rows=512):
    n, c = prediction.shape
    return pl.pallas_call(
        _dlr_body,
        grid=(n // block_rows,),
        in_specs=[
            pl.BlockSpec((block_rows, c), lambda i: (i, 0)),
            pl.BlockSpec((block_rows,), lambda i: (i,)),
        ],
        out_specs=pl.BlockSpec((block_rows,), lambda i: (i,)),
        out_shape=jax.ShapeDtypeStruct((n,), jnp.float32),
    )(prediction, y)


def kernel(prediction, y):
    return _dlr_tc(prediction, y)


# speedup vs baseline: 1.4075x; 1.0451x over previous
import functools

import jax
import jax.numpy as jnp
from jax.experimental import pallas as pl

_NEG_INF = float("-inf")
_BIG = 1 << 30
_LANES = 128


def _dlr_body(x_ref, y_ref, o_ref):
    x = x_ref[...]
    yv = y_ref[...].reshape(-1, 1)
    R, C = x.shape
    n_full = C // _LANES
    tail = C - n_full * _LANES

    col = jax.lax.broadcasted_iota(jnp.int32, (R, _LANES), 1)
    neg = jnp.full((R, _LANES), _NEG_INF, dtype=jnp.float32)

    ch = x[:, :_LANES]
    m0, m1, m2 = ch, neg, neg
    cacc = jnp.where(col == yv, ch, _NEG_INF)

    def insert(ch, k, m0, m1, m2, cacc):
        cacc = jnp.maximum(cacc, jnp.where(col == yv - (k * _LANES), ch, _NEG_INF))
        t1 = jnp.minimum(m0, ch)
        m0 = jnp.maximum(m0, ch)
        t2 = jnp.minimum(m1, t1)
        m1 = jnp.maximum(m1, t1)
        m2 = jnp.maximum(m2, t2)
        return m0, m1, m2, cacc

    for k in range(1, n_full):
        m0, m1, m2, cacc = insert(x[:, k * _LANES:(k + 1) * _LANES], k, m0, m1, m2, cacc)

    cht = jnp.concatenate([x[:, n_full * _LANES:], neg[:, : _LANES - tail]], axis=1)
    m0, m1, m2, cacc = insert(cht, n_full, m0, m1, m2, cacc)

    c = jnp.max(cacc, axis=1, keepdims=True)

    u = jnp.concatenate([m0, m1, m2], axis=1)
    ucol = jax.lax.broadcasted_iota(jnp.int32, u.shape, 1)
    p0 = jnp.max(u, axis=1, keepdims=True)
    a0 = jnp.min(jnp.where(u == p0, ucol, _BIG), axis=1, keepdims=True)
    u1 = jnp.where(ucol == a0, _NEG_INF, u)
    p1 = jnp.max(u1, axis=1, keepdims=True)
    a1 = jnp.min(jnp.where(u1 == p1, ucol, _BIG), axis=1, keepdims=True)
    u2 = jnp.where(ucol == a1, _NEG_INF, u1)
    p2 = jnp.max(u2, axis=1, keepdims=True)

    target = jnp.where(c == p0, p1, p0)
    o_ref[...] = ((target - c) / (p0 - p2)).reshape(-1)


@functools.partial(jax.jit, static_argnames=("block_rows",))
def _dlr_tc(prediction, y, block_rows=1024):
    n, c = prediction.shape
    return pl.pallas_call(
        _dlr_body,
        grid=(n // block_rows,),
        in_specs=[
            pl.BlockSpec((block_rows, c), lambda i: (i, 0)),
            pl.BlockSpec((block_rows,), lambda i: (i,)),
        ],
        out_specs=pl.BlockSpec((block_rows,), lambda i: (i,)),
        out_shape=jax.ShapeDtypeStruct((n,), jnp.float32),
    )(prediction, y)


def kernel(prediction, y):
    return _dlr_tc(prediction, y)
